# baseline (speedup 1.0000x reference)
import jax
import jax.numpy as jnp
from jax import lax
from jax.experimental import pallas as pl
from jax.experimental.pallas import tpu as pltpu
from jax.experimental.pallas import tpu_sc as plsc

T, D, FF, E = 2048, 1024, 2048, 8
H = D // 2
BLK = 128
S = T * 2 + E * BLK
NB = S // BLK
NW = 32
TW = T // NW


def _pack_halves(a):
    b = a.astype(jnp.bfloat16)
    lo = lax.bitcast_convert_type(b[:, :H], jnp.uint16).astype(jnp.uint32)
    hi = lax.bitcast_convert_type(b[:, H:], jnp.uint16).astype(jnp.uint32)
    return (lo | (hi << 16)).astype(jnp.int32)


def _unpack_halves(p):
    lo = lax.bitcast_convert_type((p & 0xFFFF).astype(jnp.uint16),
                                  jnp.bfloat16)
    hi = lax.bitcast_convert_type(
        (p >> 16).astype(jnp.uint16), jnp.bfloat16)
    return jnp.concatenate([lo, hi], axis=1)


def _router_body(x_ref, wr_ref, xpk_ref, s0_ref, s1_ref, w0_ref, w1_ref,
                 be_ref, act_ref, loss_ref):
    x = x_ref[...]
    xpk_ref[...] = _pack_halves(x)
    logits = jnp.dot(x, wr_ref[...], preferred_element_type=jnp.float32)
    m = jnp.max(logits, axis=1, keepdims=True)
    ex = jnp.exp(logits - m)
    probs = ex / jnp.sum(ex, axis=1, keepdims=True)
    lane = lax.broadcasted_iota(jnp.int32, probs.shape, 1)
    m1 = jnp.max(probs, axis=1, keepdims=True)
    i1 = jnp.min(jnp.where(probs == m1, lane, E), axis=1, keepdims=True)
    sel1 = lane == i1
    probsm = jnp.where(sel1, -jnp.inf, probs)
    m2 = jnp.max(probsm, axis=1, keepdims=True)
    i2 = jnp.min(jnp.where(probsm == m2, lane, E), axis=1, keepdims=True)
    sel2 = lane == i2

    dmask = ((sel1 & (m1 > 0.0)) | (sel2 & (m2 > 0.0))).astype(jnp.float32)
    frac = jnp.mean(dmask, axis=0)
    meanp = jnp.mean(probs, axis=0)
    loss_ref[0, 0] = jnp.float32(E) * jnp.sum(frac * meanp)

    c1 = sel1.astype(jnp.int32)
    c2 = sel2.astype(jnp.int32)
    c = c1 + c2
    inc = c
    s = 1
    while s < T:
        inc = inc + jnp.concatenate(
            [jnp.zeros((s, E), jnp.int32), inc[:T - s]], axis=0)
        s *= 2
    excl = inc - c
    counts = jnp.sum(c, axis=0, keepdims=True)
    pc = ((counts + (BLK - 1)) // BLK) * BLK
    erow = lax.broadcasted_iota(jnp.int32, (E, E), 0)
    ecol = lax.broadcasted_iota(jnp.int32, (E, E), 1)
    tri = (erow < ecol).astype(jnp.float32)
    off = jnp.dot(pc.astype(jnp.float32), tri,
                  preferred_element_type=jnp.float32).astype(jnp.int32)

    rank0 = jnp.sum(c1 * excl, axis=1, keepdims=True)
    rank1 = jnp.sum(c2 * (excl + c1), axis=1, keepdims=True)
    off0 = jnp.sum(c1 * off, axis=1, keepdims=True)
    off1 = jnp.sum(c2 * off, axis=1, keepdims=True)
    s0_ref[...] = off0 + rank0
    s1_ref[...] = off1 + rank1
    w0_ref[...] = jnp.broadcast_to(m1, (T, 128))
    w1_ref[...] = jnp.broadcast_to(m2, (T, 128))

    ends = off + pc
    bstart = lax.broadcasted_iota(jnp.int32, (NB, E), 0) * BLK
    be = jnp.minimum(jnp.sum((bstart >= ends).astype(jnp.int32),
                             axis=1, keepdims=True), E - 1)
    oh = (lax.broadcasted_iota(jnp.int32, (NB, E), 1) == be).astype(jnp.int32)
    real_end = jnp.sum(oh * (off + counts), axis=1, keepdims=True)
    bcol = lax.broadcasted_iota(jnp.int32, (NB, 1), 0) * BLK
    be_ref[...] = be
    act_ref[...] = (real_end > bcol).astype(jnp.int32)


def _router(x, wr):
    return pl.pallas_call(
        _router_body,
        out_shape=(
            jax.ShapeDtypeStruct((T, H), jnp.int32),
            jax.ShapeDtypeStruct((T, 1), jnp.int32),
            jax.ShapeDtypeStruct((T, 1), jnp.int32),
            jax.ShapeDtypeStruct((T, 128), jnp.float32),
            jax.ShapeDtypeStruct((T, 128), jnp.float32),
            jax.ShapeDtypeStruct((NB, 1), jnp.int32),
            jax.ShapeDtypeStruct((NB, 1), jnp.int32),
            jax.ShapeDtypeStruct((1, 1), jnp.float32),
        ),
        out_specs=(
            pl.BlockSpec(memory_space=pltpu.VMEM),
            pl.BlockSpec(memory_space=pltpu.VMEM),
            pl.BlockSpec(memory_space=pltpu.VMEM),
            pl.BlockSpec(memory_space=pltpu.VMEM),
            pl.BlockSpec(memory_space=pltpu.VMEM),
            pl.BlockSpec(memory_space=pltpu.VMEM),
            pl.BlockSpec(memory_space=pltpu.VMEM),
            pl.BlockSpec(memory_space=pltpu.SMEM),
        ),
    )(x, wr)


def _sc_mesh():
    return plsc.VectorSubcoreMesh(core_axis_name="c", subcore_axis_name="s",
                                  num_cores=2, num_subcores=16)


def _dispatch(xpk, slot0, slot1, w0w, w1w):
    @pl.kernel(
        out_type=(
            jax.ShapeDtypeStruct((S, H), jnp.int32),
            jax.ShapeDtypeStruct((S, 128), jnp.float32),
        ),
        mesh=_sc_mesh(),
        scratch_types=[
            pltpu.VMEM((1, TW), jnp.int32),
            pltpu.VMEM((1, TW), jnp.int32),
            pltpu.VMEM((TW, H), jnp.int32),
            pltpu.VMEM((TW, 128), jnp.float32),
        ],
    )
    def disp(x_hbm, s0_hbm, s1_hbm, w0_hbm, w1_hbm, xs_hbm, ws_hbm,
             idx0, idx1, xv, wv):
        wid = lax.axis_index("s") * 2 + lax.axis_index("c")
        base = wid * TW
        pltpu.sync_copy(s0_hbm.at[pl.ds(base, TW)], idx0.at[0])
        pltpu.sync_copy(s1_hbm.at[pl.ds(base, TW)], idx1.at[0])
        pltpu.sync_copy(x_hbm.at[pl.ds(base, TW)], xv)
        pltpu.sync_copy(xv, xs_hbm.at[idx0.at[0]])
        pltpu.sync_copy(xv, xs_hbm.at[idx1.at[0]])
        pltpu.sync_copy(w0_hbm.at[pl.ds(base, TW)], wv)
        pltpu.sync_copy(wv, ws_hbm.at[idx0.at[0]])
        pltpu.sync_copy(w1_hbm.at[pl.ds(base, TW)], wv)
        pltpu.sync_copy(wv, ws_hbm.at[idx1.at[0]])

    return disp(xpk, slot0, slot1, w0w, w1w)


def _ffn_body(be_ref, act_ref, xs_ref, w1_ref, w2_ref, ws_ref, ys_ref):
    b = pl.program_id(0)

    @pl.when(act_ref[b] > 0)
    def _():
        xb = _unpack_halves(xs_ref[...])
        w1 = w1_ref[0].astype(jnp.bfloat16)
        h = jnp.dot(xb, w1, preferred_element_type=jnp.float32)
        h = jnp.maximum(h, 0.0).astype(jnp.bfloat16)
        w2 = w2_ref[0].astype(jnp.bfloat16)
        y = jnp.dot(h, w2, preferred_element_type=jnp.float32)
        ys_ref[...] = _pack_halves(y * ws_ref[:, 0:1])


def _ffn(be, act, xs, w1, w2, ws):
    return pl.pallas_call(
        _ffn_body,
        grid_spec=pltpu.PrefetchScalarGridSpec(
            num_scalar_prefetch=2,
            grid=(NB,),
            in_specs=[
                pl.BlockSpec((BLK, H), lambda b, be, act: (b, 0)),
                pl.BlockSpec((1, D, FF), lambda b, be, act: (be[b], 0, 0)),
                pl.BlockSpec((1, FF, D), lambda b, be, act: (be[b], 0, 0)),
                pl.BlockSpec((BLK, 128), lambda b, be, act: (b, 0)),
            ],
            out_specs=pl.BlockSpec((BLK, H), lambda b, be, act: (b, 0)),
        ),
        out_shape=jax.ShapeDtypeStruct((S, H), jnp.int32),
        compiler_params=pltpu.CompilerParams(
            dimension_semantics=("arbitrary",),
        ),
    )(be, act, xs, w1, w2, ws)


_CW = 32


def _combine(slot0, slot1, ys):
    @pl.kernel(
        out_type=jax.ShapeDtypeStruct((T, D), jnp.float32),
        mesh=_sc_mesh(),
        scratch_types=[
            pltpu.VMEM((1, _CW), jnp.int32),
            pltpu.VMEM((1, _CW), jnp.int32),
            pltpu.VMEM((_CW, H), jnp.int32),
            pltpu.VMEM((_CW, H), jnp.int32),
            pltpu.VMEM((_CW, D), jnp.float32),
            pltpu.SemaphoreType.DMA,
            pltpu.SemaphoreType.DMA,
        ],
    )
    def comb(s0_hbm, s1_hbm, ys_hbm, out_hbm, idx0, idx1, g0, g1, ov,
             sem0, sem1):
        wid = lax.axis_index("s") * 2 + lax.axis_index("c")
        himask = jnp.int32(-65536)

        @pl.loop(0, TW // _CW)
        def _(sc):
            base = wid * TW + sc * _CW
            pltpu.sync_copy(s0_hbm.at[pl.ds(base, _CW)], idx0.at[0])
            pltpu.sync_copy(s1_hbm.at[pl.ds(base, _CW)], idx1.at[0])
            cp0 = pltpu.async_copy(ys_hbm.at[idx0.at[0]], g0, sem0)
            cp1 = pltpu.async_copy(ys_hbm.at[idx1.at[0]], g1, sem1)
            cp0.wait()
            cp1.wait()

            @pl.loop(0, _CW)
            def _(i):
                @pl.loop(0, H, step=16)
                def _(cc):
                    slc = (pl.ds(i, 1), pl.ds(cc, 16))
                    c0 = g0.at[*slc][...]
                    c1 = g1.at[*slc][...]
                    lo = (lax.bitcast_convert_type(c0 << 16, jnp.float32)
                          + lax.bitcast_convert_type(c1 << 16, jnp.float32))
                    hi = (lax.bitcast_convert_type(c0 & himask, jnp.float32)
                          + lax.bitcast_convert_type(c1 & himask, jnp.float32))
                    ov.at[pl.ds(i, 1), pl.ds(cc, 16)][...] = lo
                    ov.at[pl.ds(i, 1), pl.ds(H + cc, 16)][...] = hi

            pltpu.sync_copy(ov, out_hbm.at[pl.ds(base, _CW)])

    return comb(slot0, slot1, ys)


def kernel(input_batch, W_router, W1, W2):
    x = input_batch
    xpk, s0, s1, w0w, w1w, be, act, loss = _router(x, W_router)
    slot0 = s0.reshape(T)
    slot1 = s1.reshape(T)
    xs, ws = _dispatch(xpk, slot0, slot1, w0w, w1w)
    ys = _ffn(be.reshape(NB), act.reshape(NB), xs, W1, W2, ws)
    out = _combine(slot0, slot1, ys)
    return (out, loss.reshape(()))


# speedup vs baseline: 1.0662x; 1.0662x over previous
import jax
import jax.numpy as jnp
from jax import lax
from jax.experimental import pallas as pl
from jax.experimental.pallas import tpu as pltpu
from jax.experimental.pallas import tpu_sc as plsc

T, D, FF, E = 2048, 1024, 2048, 8
H = D // 2
BLK = 256
S = T * 2 + E * BLK
NB = S // BLK
NW = 32
TW = T // NW


def _pack_halves(a):
    b = a.astype(jnp.bfloat16)
    lo = lax.bitcast_convert_type(b[:, :H], jnp.uint16).astype(jnp.uint32)
    hi = lax.bitcast_convert_type(b[:, H:], jnp.uint16).astype(jnp.uint32)
    return (lo | (hi << 16)).astype(jnp.int32)


def _unpack_halves(p):
    lo = lax.bitcast_convert_type((p & 0xFFFF).astype(jnp.uint16),
                                  jnp.bfloat16)
    hi = lax.bitcast_convert_type(
        (p >> 16).astype(jnp.uint16), jnp.bfloat16)
    return jnp.concatenate([lo, hi], axis=1)


def _router_body(x_ref, wr_ref, xpk_ref, s0_ref, s1_ref, w0_ref, w1_ref,
                 be_ref, act_ref, loss_ref):
    x = x_ref[...]
    xpk_ref[...] = _pack_halves(x)
    logits = jnp.dot(x, wr_ref[...], preferred_element_type=jnp.float32)
    m = jnp.max(logits, axis=1, keepdims=True)
    ex = jnp.exp(logits - m)
    probs = ex / jnp.sum(ex, axis=1, keepdims=True)
    lane = lax.broadcasted_iota(jnp.int32, probs.shape, 1)
    m1 = jnp.max(probs, axis=1, keepdims=True)
    i1 = jnp.min(jnp.where(probs == m1, lane, E), axis=1, keepdims=True)
    sel1 = lane == i1
    probsm = jnp.where(sel1, -jnp.inf, probs)
    m2 = jnp.max(probsm, axis=1, keepdims=True)
    i2 = jnp.min(jnp.where(probsm == m2, lane, E), axis=1, keepdims=True)
    sel2 = lane == i2

    dmask = ((sel1 & (m1 > 0.0)) | (sel2 & (m2 > 0.0))).astype(jnp.float32)
    frac = jnp.mean(dmask, axis=0)
    meanp = jnp.mean(probs, axis=0)
    loss_ref[0, 0] = jnp.float32(E) * jnp.sum(frac * meanp)

    c1 = sel1.astype(jnp.int32)
    c2 = sel2.astype(jnp.int32)
    c = c1 + c2
    inc = c
    s = 1
    while s < T:
        inc = inc + jnp.concatenate(
            [jnp.zeros((s, E), jnp.int32), inc[:T - s]], axis=0)
        s *= 2
    excl = inc - c
    counts = jnp.sum(c, axis=0, keepdims=True)
    pc = ((counts + (BLK - 1)) // BLK) * BLK
    erow = lax.broadcasted_iota(jnp.int32, (E, E), 0)
    ecol = lax.broadcasted_iota(jnp.int32, (E, E), 1)
    tri = (erow < ecol).astype(jnp.float32)
    off = jnp.dot(pc.astype(jnp.float32), tri,
                  preferred_element_type=jnp.float32).astype(jnp.int32)

    rank0 = jnp.sum(c1 * excl, axis=1, keepdims=True)
    rank1 = jnp.sum(c2 * (excl + c1), axis=1, keepdims=True)
    off0 = jnp.sum(c1 * off, axis=1, keepdims=True)
    off1 = jnp.sum(c2 * off, axis=1, keepdims=True)
    s0_ref[...] = off0 + rank0
    s1_ref[...] = off1 + rank1
    w0_ref[...] = jnp.broadcast_to(m1, (T, 128))
    w1_ref[...] = jnp.broadcast_to(m2, (T, 128))

    ends = off + pc
    bstart = lax.broadcasted_iota(jnp.int32, (NB, E), 0) * BLK
    be = jnp.minimum(jnp.sum((bstart >= ends).astype(jnp.int32),
                             axis=1, keepdims=True), E - 1)
    oh = (lax.broadcasted_iota(jnp.int32, (NB, E), 1) == be).astype(jnp.int32)
    real_end = jnp.sum(oh * (off + counts), axis=1, keepdims=True)
    bcol = lax.broadcasted_iota(jnp.int32, (NB, 1), 0) * BLK
    be_ref[...] = be
    act_ref[...] = (real_end > bcol).astype(jnp.int32)


def _router(x, wr):
    return pl.pallas_call(
        _router_body,
        out_shape=(
            jax.ShapeDtypeStruct((T, H), jnp.int32),
            jax.ShapeDtypeStruct((T, 1), jnp.int32),
            jax.ShapeDtypeStruct((T, 1), jnp.int32),
            jax.ShapeDtypeStruct((T, 128), jnp.float32),
            jax.ShapeDtypeStruct((T, 128), jnp.float32),
            jax.ShapeDtypeStruct((NB, 1), jnp.int32),
            jax.ShapeDtypeStruct((NB, 1), jnp.int32),
            jax.ShapeDtypeStruct((1, 1), jnp.float32),
        ),
        out_specs=(
            pl.BlockSpec(memory_space=pltpu.VMEM),
            pl.BlockSpec(memory_space=pltpu.VMEM),
            pl.BlockSpec(memory_space=pltpu.VMEM),
            pl.BlockSpec(memory_space=pltpu.VMEM),
            pl.BlockSpec(memory_space=pltpu.VMEM),
            pl.BlockSpec(memory_space=pltpu.VMEM),
            pl.BlockSpec(memory_space=pltpu.VMEM),
            pl.BlockSpec(memory_space=pltpu.SMEM),
        ),
    )(x, wr)


def _sc_mesh():
    return plsc.VectorSubcoreMesh(core_axis_name="c", subcore_axis_name="s",
                                  num_cores=2, num_subcores=16)


def _dispatch(xpk, slot0, slot1, w0w, w1w):
    @pl.kernel(
        out_type=(
            jax.ShapeDtypeStruct((S, H), jnp.int32),
            jax.ShapeDtypeStruct((S, 128), jnp.float32),
        ),
        mesh=_sc_mesh(),
        scratch_types=[
            pltpu.VMEM((1, TW), jnp.int32),
            pltpu.VMEM((1, TW), jnp.int32),
            pltpu.VMEM((TW, H), jnp.int32),
            pltpu.VMEM((TW, 128), jnp.float32),
            pltpu.VMEM((TW, 128), jnp.float32),
            pltpu.SemaphoreType.DMA,
            pltpu.SemaphoreType.DMA,
        ],
    )
    def disp(x_hbm, s0_hbm, s1_hbm, w0_hbm, w1_hbm, xs_hbm, ws_hbm,
             idx0, idx1, xv, wv0, wv1, lsem, ssem):
        wid = lax.axis_index("s") * 2 + lax.axis_index("c")
        base = wid * TW
        l0 = pltpu.async_copy(s0_hbm.at[pl.ds(base, TW)], idx0.at[0], lsem)
        l1 = pltpu.async_copy(s1_hbm.at[pl.ds(base, TW)], idx1.at[0], lsem)
        l2 = pltpu.async_copy(x_hbm.at[pl.ds(base, TW)], xv, lsem)
        l3 = pltpu.async_copy(w0_hbm.at[pl.ds(base, TW)], wv0, lsem)
        l4 = pltpu.async_copy(w1_hbm.at[pl.ds(base, TW)], wv1, lsem)
        l0.wait()
        l1.wait()
        l2.wait()
        l3.wait()
        l4.wait()
        s0 = pltpu.async_copy(xv, xs_hbm.at[idx0.at[0]], ssem)
        s1 = pltpu.async_copy(xv, xs_hbm.at[idx1.at[0]], ssem)
        s2 = pltpu.async_copy(wv0, ws_hbm.at[idx0.at[0]], ssem)
        s3 = pltpu.async_copy(wv1, ws_hbm.at[idx1.at[0]], ssem)
        s0.wait()
        s1.wait()
        s2.wait()
        s3.wait()

    return disp(xpk, slot0, slot1, w0w, w1w)


def _ffn_body(be_ref, act_ref, xs_ref, w1_ref, w2_ref, ws_ref, ys_ref):
    b = pl.program_id(0)

    @pl.when(act_ref[b] > 0)
    def _():
        xb = _unpack_halves(xs_ref[...])
        w1 = w1_ref[0].astype(jnp.bfloat16)
        h = jnp.dot(xb, w1, preferred_element_type=jnp.float32)
        h = jnp.maximum(h, 0.0).astype(jnp.bfloat16)
        w2 = w2_ref[0].astype(jnp.bfloat16)
        y = jnp.dot(h, w2, preferred_element_type=jnp.float32)
        ys_ref[...] = _pack_halves(y * ws_ref[:, 0:1])


def _ffn(be, act, xs, w1, w2, ws):
    return pl.pallas_call(
        _ffn_body,
        grid_spec=pltpu.PrefetchScalarGridSpec(
            num_scalar_prefetch=2,
            grid=(NB,),
            in_specs=[
                pl.BlockSpec((BLK, H), lambda b, be, act: (b, 0)),
                pl.BlockSpec((1, D, FF), lambda b, be, act: (be[b], 0, 0)),
                pl.BlockSpec((1, FF, D), lambda b, be, act: (be[b], 0, 0)),
                pl.BlockSpec((BLK, 128), lambda b, be, act: (b, 0)),
            ],
            out_specs=pl.BlockSpec((BLK, H), lambda b, be, act: (b, 0)),
        ),
        out_shape=jax.ShapeDtypeStruct((S, H), jnp.int32),
        compiler_params=pltpu.CompilerParams(
            dimension_semantics=("arbitrary",),
        ),
    )(be, act, xs, w1, w2, ws)


_CW = 16


def _combine(slot0, slot1, ys):
    nch = TW // _CW

    @pl.kernel(
        out_type=jax.ShapeDtypeStruct((T, D), jnp.float32),
        mesh=_sc_mesh(),
        scratch_types=[
            pltpu.VMEM((1, _CW), jnp.int32),
            pltpu.VMEM((1, _CW), jnp.int32),
            pltpu.VMEM((1, _CW), jnp.int32),
            pltpu.VMEM((1, _CW), jnp.int32),
            pltpu.VMEM((_CW, H), jnp.int32),
            pltpu.VMEM((_CW, H), jnp.int32),
            pltpu.VMEM((_CW, H), jnp.int32),
            pltpu.VMEM((_CW, H), jnp.int32),
            pltpu.VMEM((_CW, D), jnp.float32),
            pltpu.SemaphoreType.DMA,
            pltpu.SemaphoreType.DMA,
        ],
    )
    def comb(s0_hbm, s1_hbm, ys_hbm, out_hbm, idx0a, idx1a, idx0b, idx1b,
             g0a, g1a, g0b, g1b, ov, sema, semb):
        wid = lax.axis_index("s") * 2 + lax.axis_index("c")
        himask = jnp.int32(-65536)
        slots = ((idx0a, idx1a, g0a, g1a, sema),
                 (idx0b, idx1b, g0b, g1b, semb))

        def start(sc, slot):
            i0, i1, g0, g1, sem = slot
            base = wid * TW + sc * _CW
            pltpu.sync_copy(s0_hbm.at[pl.ds(base, _CW)], i0.at[0])
            pltpu.sync_copy(s1_hbm.at[pl.ds(base, _CW)], i1.at[0])
            cp0 = pltpu.async_copy(ys_hbm.at[i0.at[0]], g0, sem)
            cp1 = pltpu.async_copy(ys_hbm.at[i1.at[0]], g1, sem)
            return cp0, cp1

        pend = start(0, slots[0])
        for sc in range(nch):
            _, _, g0, g1, _ = slots[sc % 2]
            pend[0].wait()
            pend[1].wait()
            if sc + 1 < nch:
                pend = start(sc + 1, slots[(sc + 1) % 2])

            @pl.loop(0, _CW)
            def _(i):
                @pl.loop(0, H, step=16)
                def _(cc):
                    slc = (pl.ds(i, 1), pl.ds(cc, 16))
                    c0 = g0.at[*slc][...]
                    c1 = g1.at[*slc][...]
                    lo = (lax.bitcast_convert_type(c0 << 16, jnp.float32)
                          + lax.bitcast_convert_type(c1 << 16, jnp.float32))
                    hi = (lax.bitcast_convert_type(c0 & himask, jnp.float32)
                          + lax.bitcast_convert_type(c1 & himask, jnp.float32))
                    ov.at[pl.ds(i, 1), pl.ds(cc, 16)][...] = lo
                    ov.at[pl.ds(i, 1), pl.ds(H + cc, 16)][...] = hi

            pltpu.sync_copy(ov, out_hbm.at[pl.ds(wid * TW + sc * _CW, _CW)])

    return comb(slot0, slot1, ys)


def kernel(input_batch, W_router, W1, W2):
    x = input_batch
    xpk, s0, s1, w0w, w1w, be, act, loss = _router(x, W_router)
    slot0 = s0.reshape(T)
    slot1 = s1.reshape(T)
    xs, ws = _dispatch(xpk, slot0, slot1, w0w, w1w)
    ys = _ffn(be.reshape(NB), act.reshape(NB), xs, W1, W2, ws)
    out = _combine(slot0, slot1, ys)
    return (out, loss.reshape(()))


# speedup vs baseline: 1.1911x; 1.1171x over previous
import jax
import jax.numpy as jnp
from jax import lax
from jax.experimental import pallas as pl
from jax.experimental.pallas import tpu as pltpu
from jax.experimental.pallas import tpu_sc as plsc

T, D, FF, E = 2048, 1024, 2048, 8
H = D // 2
BLK = 256
S = T * 2 + E * BLK
NB = S // BLK
NW = 32
TW = T // NW


def _pack_halves(a):
    b = a.astype(jnp.bfloat16)
    lo = lax.bitcast_convert_type(b[:, :H], jnp.uint16).astype(jnp.uint32)
    hi = lax.bitcast_convert_type(b[:, H:], jnp.uint16).astype(jnp.uint32)
    return (lo | (hi << 16)).astype(jnp.int32)


def _unpack_halves(p):
    lo = lax.bitcast_convert_type((p & 0xFFFF).astype(jnp.uint16),
                                  jnp.bfloat16)
    hi = lax.bitcast_convert_type(
        (p >> 16).astype(jnp.uint16), jnp.bfloat16)
    return jnp.concatenate([lo, hi], axis=1)


def _router_body(x_ref, wr_ref, xpk_ref, s0_ref, s1_ref, w0_ref, w1_ref,
                 be_ref, act_ref, chg_ref, slt_ref, nsl_ref, hnx_ref,
                 nxe_ref, loss_ref):
    x = x_ref[...]
    xpk_ref[...] = _pack_halves(x)
    logits = jnp.dot(x, wr_ref[...], preferred_element_type=jnp.float32)
    m = jnp.max(logits, axis=1, keepdims=True)
    ex = jnp.exp(logits - m)
    probs = ex / jnp.sum(ex, axis=1, keepdims=True)
    lane = lax.broadcasted_iota(jnp.int32, probs.shape, 1)
    m1 = jnp.max(probs, axis=1, keepdims=True)
    i1 = jnp.min(jnp.where(probs == m1, lane, E), axis=1, keepdims=True)
    sel1 = lane == i1
    probsm = jnp.where(sel1, -jnp.inf, probs)
    m2 = jnp.max(probsm, axis=1, keepdims=True)
    i2 = jnp.min(jnp.where(probsm == m2, lane, E), axis=1, keepdims=True)
    sel2 = lane == i2

    dmask = ((sel1 & (m1 > 0.0)) | (sel2 & (m2 > 0.0))).astype(jnp.float32)
    frac = jnp.mean(dmask, axis=0)
    meanp = jnp.mean(probs, axis=0)
    loss_ref[0, 0] = jnp.float32(E) * jnp.sum(frac * meanp)

    c1 = sel1.astype(jnp.int32)
    c2 = sel2.astype(jnp.int32)
    c = c1 + c2
    inc = c
    s = 1
    while s < T:
        inc = inc + jnp.concatenate(
            [jnp.zeros((s, E), jnp.int32), inc[:T - s]], axis=0)
        s *= 2
    excl = inc - c
    counts = jnp.sum(c, axis=0, keepdims=True)
    pc = ((counts + (BLK - 1)) // BLK) * BLK
    erow = lax.broadcasted_iota(jnp.int32, (E, E), 0)
    ecol = lax.broadcasted_iota(jnp.int32, (E, E), 1)
    tri = (erow < ecol).astype(jnp.float32)
    off = jnp.dot(pc.astype(jnp.float32), tri,
                  preferred_element_type=jnp.float32).astype(jnp.int32)

    rank0 = jnp.sum(c1 * excl, axis=1, keepdims=True)
    rank1 = jnp.sum(c2 * (excl + c1), axis=1, keepdims=True)
    off0 = jnp.sum(c1 * off, axis=1, keepdims=True)
    off1 = jnp.sum(c2 * off, axis=1, keepdims=True)
    s0_ref[...] = off0 + rank0
    s1_ref[...] = off1 + rank1
    w0_ref[...] = jnp.broadcast_to(m1, (T, 128))
    w1_ref[...] = jnp.broadcast_to(m2, (T, 128))

    ends = off + pc
    bstart = lax.broadcasted_iota(jnp.int32, (NB, E), 0) * BLK
    be = jnp.minimum(jnp.sum((bstart >= ends).astype(jnp.int32),
                             axis=1, keepdims=True), E - 1)
    oh = (lax.broadcasted_iota(jnp.int32, (NB, E), 1) == be).astype(jnp.int32)
    real_end = jnp.sum(oh * (off + counts), axis=1, keepdims=True)
    bcol = lax.broadcasted_iota(jnp.int32, (NB, 1), 0) * BLK
    be_ref[...] = be
    act_ref[...] = (real_end > bcol).astype(jnp.int32)

    bprev = jnp.concatenate([be[:1], be[:NB - 1]], axis=0)
    biota = lax.broadcasted_iota(jnp.int32, (NB, 1), 0)
    chg = ((biota == 0) | (be != bprev)).astype(jnp.int32)
    r = chg
    s = 1
    while s < NB:
        r = r + jnp.concatenate(
            [jnp.zeros((s, 1), jnp.int32), r[:NB - s]], axis=0)
        s *= 2
    BIG = jnp.int32(1 << 30)
    enc = jnp.where(chg == 1, biota * 16 + be, BIG)
    suf = jnp.concatenate([enc[1:], jnp.full((1, 1), BIG, jnp.int32)],
                          axis=0)
    s = 1
    while s < NB:
        suf = jnp.minimum(suf, jnp.concatenate(
            [suf[s:], jnp.full((s, 1), BIG, jnp.int32)], axis=0))
        s *= 2
    chg_ref[...] = chg
    slt_ref[...] = (r - 1) % 2
    nsl_ref[...] = r % 2
    hnx_ref[...] = (suf < BIG).astype(jnp.int32)
    nxe_ref[...] = jnp.where(suf < BIG, suf % 16, 0)


def _router(x, wr):
    return pl.pallas_call(
        _router_body,
        out_shape=(
            jax.ShapeDtypeStruct((T, H), jnp.int32),
            jax.ShapeDtypeStruct((T, 1), jnp.int32),
            jax.ShapeDtypeStruct((T, 1), jnp.int32),
            jax.ShapeDtypeStruct((T, 128), jnp.float32),
            jax.ShapeDtypeStruct((T, 128), jnp.float32),
            jax.ShapeDtypeStruct((NB, 1), jnp.int32),
            jax.ShapeDtypeStruct((NB, 1), jnp.int32),
            jax.ShapeDtypeStruct((NB, 1), jnp.int32),
            jax.ShapeDtypeStruct((NB, 1), jnp.int32),
            jax.ShapeDtypeStruct((NB, 1), jnp.int32),
            jax.ShapeDtypeStruct((NB, 1), jnp.int32),
            jax.ShapeDtypeStruct((NB, 1), jnp.int32),
            jax.ShapeDtypeStruct((1, 1), jnp.float32),
        ),
        out_specs=(
            pl.BlockSpec(memory_space=pltpu.VMEM),
            pl.BlockSpec(memory_space=pltpu.VMEM),
            pl.BlockSpec(memory_space=pltpu.VMEM),
            pl.BlockSpec(memory_space=pltpu.VMEM),
            pl.BlockSpec(memory_space=pltpu.VMEM),
            pl.BlockSpec(memory_space=pltpu.VMEM),
            pl.BlockSpec(memory_space=pltpu.VMEM),
            pl.BlockSpec(memory_space=pltpu.VMEM),
            pl.BlockSpec(memory_space=pltpu.VMEM),
            pl.BlockSpec(memory_space=pltpu.VMEM),
            pl.BlockSpec(memory_space=pltpu.VMEM),
            pl.BlockSpec(memory_space=pltpu.VMEM),
            pl.BlockSpec(memory_space=pltpu.SMEM),
        ),
    )(x, wr)


def _sc_mesh():
    return plsc.VectorSubcoreMesh(core_axis_name="c", subcore_axis_name="s",
                                  num_cores=2, num_subcores=16)


def _dispatch(xpk, slot0, slot1, w0w, w1w):
    @pl.kernel(
        out_type=(
            jax.ShapeDtypeStruct((S, H), jnp.int32),
            jax.ShapeDtypeStruct((S, 128), jnp.float32),
        ),
        mesh=_sc_mesh(),
        scratch_types=[
            pltpu.VMEM((1, TW), jnp.int32),
            pltpu.VMEM((1, TW), jnp.int32),
            pltpu.VMEM((TW, H), jnp.int32),
            pltpu.VMEM((TW, 128), jnp.float32),
            pltpu.VMEM((TW, 128), jnp.float32),
            pltpu.SemaphoreType.DMA,
            pltpu.SemaphoreType.DMA,
        ],
    )
    def disp(x_hbm, s0_hbm, s1_hbm, w0_hbm, w1_hbm, xs_hbm, ws_hbm,
             idx0, idx1, xv, wv0, wv1, lsem, ssem):
        wid = lax.axis_index("s") * 2 + lax.axis_index("c")
        base = wid * TW
        l0 = pltpu.async_copy(s0_hbm.at[pl.ds(base, TW)], idx0.at[0], lsem)
        l1 = pltpu.async_copy(s1_hbm.at[pl.ds(base, TW)], idx1.at[0], lsem)
        l2 = pltpu.async_copy(x_hbm.at[pl.ds(base, TW)], xv, lsem)
        l3 = pltpu.async_copy(w0_hbm.at[pl.ds(base, TW)], wv0, lsem)
        l4 = pltpu.async_copy(w1_hbm.at[pl.ds(base, TW)], wv1, lsem)
        l0.wait()
        l1.wait()
        l2.wait()
        l3.wait()
        l4.wait()
        s0 = pltpu.async_copy(xv, xs_hbm.at[idx0.at[0]], ssem)
        s1 = pltpu.async_copy(xv, xs_hbm.at[idx1.at[0]], ssem)
        s2 = pltpu.async_copy(wv0, ws_hbm.at[idx0.at[0]], ssem)
        s3 = pltpu.async_copy(wv1, ws_hbm.at[idx1.at[0]], ssem)
        s0.wait()
        s1.wait()
        s2.wait()
        s3.wait()

    return disp(xpk, slot0, slot1, w0w, w1w)


def _ffn_body(be_r, act_r, chg_r, slt_r, nsl_r, hnx_r, nxe_r,
              xs_ref, w1_hbm, w2_hbm, ws_ref, ys_ref, w1s, w2s, sw1, sw2):
    b = pl.program_id(0)

    def cp1(e, s):
        return pltpu.make_async_copy(w1_hbm.at[e], w1s.at[s], sw1.at[s])

    def cp2(e, s):
        return pltpu.make_async_copy(w2_hbm.at[e], w2s.at[s], sw2.at[s])

    @pl.when(b == 0)
    def _():
        cp1(be_r[0], 0).start()
        cp2(be_r[0], 0).start()

    @pl.when(chg_r[b] == 1)
    def _():
        s = slt_r[b]
        cp1(be_r[b], s).wait()
        cp2(be_r[b], s).wait()

        @pl.when(hnx_r[b] == 1)
        def _():
            cp1(nxe_r[b], nsl_r[b]).start()
            cp2(nxe_r[b], nsl_r[b]).start()

    @pl.when(act_r[b] > 0)
    def _():
        s = slt_r[b]
        xb = _unpack_halves(xs_ref[...])
        w1 = w1s[s].astype(jnp.bfloat16)
        h = jnp.dot(xb, w1, preferred_element_type=jnp.float32)
        h = jnp.maximum(h, 0.0).astype(jnp.bfloat16)
        w2 = w2s[s].astype(jnp.bfloat16)
        y = jnp.dot(h, w2, preferred_element_type=jnp.float32)
        ys_ref[...] = _pack_halves(y * ws_ref[:, 0:1])


def _ffn(be, act, chg, slt, nsl, hnx, nxe, xs, w1, w2, ws):
    return pl.pallas_call(
        _ffn_body,
        grid_spec=pltpu.PrefetchScalarGridSpec(
            num_scalar_prefetch=7,
            grid=(NB,),
            in_specs=[
                pl.BlockSpec((BLK, H), lambda b, *_: (b, 0)),
                pl.BlockSpec(memory_space=pl.ANY),
                pl.BlockSpec(memory_space=pl.ANY),
                pl.BlockSpec((BLK, 128), lambda b, *_: (b, 0)),
            ],
            out_specs=pl.BlockSpec((BLK, H), lambda b, *_: (b, 0)),
            scratch_shapes=[
                pltpu.VMEM((2, D, FF), jnp.float32),
                pltpu.VMEM((2, FF, D), jnp.float32),
                pltpu.SemaphoreType.DMA((2,)),
                pltpu.SemaphoreType.DMA((2,)),
            ],
        ),
        out_shape=jax.ShapeDtypeStruct((S, H), jnp.int32),
        compiler_params=pltpu.CompilerParams(
            dimension_semantics=("arbitrary",),
        ),
    )(be, act, chg, slt, nsl, hnx, nxe, xs, w1, w2, ws)


_CW = 16


def _combine(slot0, slot1, ys):
    nch = TW // _CW

    @pl.kernel(
        out_type=jax.ShapeDtypeStruct((T, D), jnp.float32),
        mesh=_sc_mesh(),
        scratch_types=[
            pltpu.VMEM((1, _CW), jnp.int32),
            pltpu.VMEM((1, _CW), jnp.int32),
            pltpu.VMEM((1, _CW), jnp.int32),
            pltpu.VMEM((1, _CW), jnp.int32),
            pltpu.VMEM((_CW, H), jnp.int32),
            pltpu.VMEM((_CW, H), jnp.int32),
            pltpu.VMEM((_CW, H), jnp.int32),
            pltpu.VMEM((_CW, H), jnp.int32),
            pltpu.VMEM((_CW, D), jnp.float32),
            pltpu.SemaphoreType.DMA,
            pltpu.SemaphoreType.DMA,
        ],
    )
    def comb(s0_hbm, s1_hbm, ys_hbm, out_hbm, idx0a, idx1a, idx0b, idx1b,
             g0a, g1a, g0b, g1b, ov, sema, semb):
        wid = lax.axis_index("s") * 2 + lax.axis_index("c")
        himask = jnp.int32(-65536)
        slots = ((idx0a, idx1a, g0a, g1a, sema),
                 (idx0b, idx1b, g0b, g1b, semb))

        def start(sc, slot):
            i0, i1, g0, g1, sem = slot
            base = wid * TW + sc * _CW
            pltpu.sync_copy(s0_hbm.at[pl.ds(base, _CW)], i0.at[0])
            pltpu.sync_copy(s1_hbm.at[pl.ds(base, _CW)], i1.at[0])
            cp0 = pltpu.async_copy(ys_hbm.at[i0.at[0]], g0, sem)
            cp1 = pltpu.async_copy(ys_hbm.at[i1.at[0]], g1, sem)
            return cp0, cp1

        pend = start(0, slots[0])
        for sc in range(nch):
            _, _, g0, g1, _ = slots[sc % 2]
            pend[0].wait()
            pend[1].wait()
            if sc + 1 < nch:
                pend = start(sc + 1, slots[(sc + 1) % 2])

            @pl.loop(0, _CW)
            def _(i):
                @pl.loop(0, H, step=16)
                def _(cc):
                    slc = (pl.ds(i, 1), pl.ds(cc, 16))
                    c0 = g0.at[*slc][...]
                    c1 = g1.at[*slc][...]
                    lo = (lax.bitcast_convert_type(c0 << 16, jnp.float32)
                          + lax.bitcast_convert_type(c1 << 16, jnp.float32))
                    hi = (lax.bitcast_convert_type(c0 & himask, jnp.float32)
                          + lax.bitcast_convert_type(c1 & himask, jnp.float32))
                    ov.at[pl.ds(i, 1), pl.ds(cc, 16)][...] = lo
                    ov.at[pl.ds(i, 1), pl.ds(H + cc, 16)][...] = hi

            pltpu.sync_copy(ov, out_hbm.at[pl.ds(wid * TW + sc * _CW, _CW)])

    return comb(slot0, slot1, ys)


def kernel(input_batch, W_router, W1, W2):
    x = input_batch
    (xpk, s0, s1, w0w, w1w, be, act, chg, slt, nsl, hnx, nxe,
     loss) = _router(x, W_router)
    slot0 = s0.reshape(T)
    slot1 = s1.reshape(T)
    xs, ws = _dispatch(xpk, slot0, slot1, w0w, w1w)
    ys = _ffn(be.reshape(NB), act.reshape(NB), chg.reshape(NB),
              slt.reshape(NB), nsl.reshape(NB), hnx.reshape(NB),
              nxe.reshape(NB), xs, W1, W2, ws)
    out = _combine(slot0, slot1, ys)
    return (out, loss.reshape(()))


# speedup vs baseline: 1.2690x; 1.0654x over previous
import jax
import jax.numpy as jnp
from jax import lax
from jax.experimental import pallas as pl
from jax.experimental.pallas import tpu as pltpu
from jax.experimental.pallas import tpu_sc as plsc

T, D, FF, E = 2048, 1024, 2048, 8
H = D // 2
BLK = 256
S = T * 2 + E * BLK
NB = S // BLK
NW = 32
TW = T // NW


def _pack_halves(a):
    b = a.astype(jnp.bfloat16)
    lo = lax.bitcast_convert_type(b[:, :H], jnp.uint16).astype(jnp.uint32)
    hi = lax.bitcast_convert_type(b[:, H:], jnp.uint16).astype(jnp.uint32)
    return (lo | (hi << 16)).astype(jnp.int32)


def _unpack_halves(p):
    lo = lax.bitcast_convert_type((p & 0xFFFF).astype(jnp.uint16),
                                  jnp.bfloat16)
    hi = lax.bitcast_convert_type(
        (p >> 16).astype(jnp.uint16), jnp.bfloat16)
    return jnp.concatenate([lo, hi], axis=1)


def _router_body(x_ref, wr_ref, xpk_ref, s0_ref, s1_ref, w0_ref, w1_ref,
                 be_ref, act_ref, chg_ref, slt_ref, nsl_ref, hnx_ref,
                 nxe_ref, loss_ref):
    x = x_ref[...]
    xpk_ref[...] = _pack_halves(x)
    logits = jnp.dot(x, wr_ref[...], preferred_element_type=jnp.float32)
    m = jnp.max(logits, axis=1, keepdims=True)
    ex = jnp.exp(logits - m)
    probs = ex / jnp.sum(ex, axis=1, keepdims=True)
    lane = lax.broadcasted_iota(jnp.int32, probs.shape, 1)
    m1 = jnp.max(probs, axis=1, keepdims=True)
    i1 = jnp.min(jnp.where(probs == m1, lane, E), axis=1, keepdims=True)
    sel1 = lane == i1
    probsm = jnp.where(sel1, -jnp.inf, probs)
    m2 = jnp.max(probsm, axis=1, keepdims=True)
    i2 = jnp.min(jnp.where(probsm == m2, lane, E), axis=1, keepdims=True)
    sel2 = lane == i2

    dmask = ((sel1 & (m1 > 0.0)) | (sel2 & (m2 > 0.0))).astype(jnp.float32)
    frac = jnp.mean(dmask, axis=0)
    meanp = jnp.mean(probs, axis=0)
    loss_ref[0, 0] = jnp.float32(E) * jnp.sum(frac * meanp)

    c1 = sel1.astype(jnp.int32)
    c2 = sel2.astype(jnp.int32)
    c = c1 + c2
    inc = c
    s = 1
    while s < T:
        inc = inc + jnp.concatenate(
            [jnp.zeros((s, E), jnp.int32), inc[:T - s]], axis=0)
        s *= 2
    excl = inc - c
    counts = jnp.sum(c, axis=0, keepdims=True)
    pc = ((counts + (BLK - 1)) // BLK) * BLK
    erow = lax.broadcasted_iota(jnp.int32, (E, E), 0)
    ecol = lax.broadcasted_iota(jnp.int32, (E, E), 1)
    tri = (erow < ecol).astype(jnp.float32)
    off = jnp.dot(pc.astype(jnp.float32), tri,
                  preferred_element_type=jnp.float32).astype(jnp.int32)

    rank0 = jnp.sum(c1 * excl, axis=1, keepdims=True)
    rank1 = jnp.sum(c2 * (excl + c1), axis=1, keepdims=True)
    off0 = jnp.sum(c1 * off, axis=1, keepdims=True)
    off1 = jnp.sum(c2 * off, axis=1, keepdims=True)
    s0_ref[...] = off0 + rank0
    s1_ref[...] = off1 + rank1
    w0_ref[...] = jnp.broadcast_to(m1, (T, 128))
    w1_ref[...] = jnp.broadcast_to(m2, (T, 128))

    ends = off + pc
    bstart = lax.broadcasted_iota(jnp.int32, (NB, E), 0) * BLK
    be = jnp.minimum(jnp.sum((bstart >= ends).astype(jnp.int32),
                             axis=1, keepdims=True), E - 1)
    oh = (lax.broadcasted_iota(jnp.int32, (NB, E), 1) == be).astype(jnp.int32)
    real_end = jnp.sum(oh * (off + counts), axis=1, keepdims=True)
    bcol = lax.broadcasted_iota(jnp.int32, (NB, 1), 0) * BLK
    be_ref[...] = be
    act_ref[...] = (real_end > bcol).astype(jnp.int32)

    bprev = jnp.concatenate([be[:1], be[:NB - 1]], axis=0)
    biota = lax.broadcasted_iota(jnp.int32, (NB, 1), 0)
    chg = ((biota == 0) | (be != bprev)).astype(jnp.int32)
    r = chg
    s = 1
    while s < NB:
        r = r + jnp.concatenate(
            [jnp.zeros((s, 1), jnp.int32), r[:NB - s]], axis=0)
        s *= 2
    BIG = jnp.int32(1 << 30)
    enc = jnp.where(chg == 1, biota * 16 + be, BIG)
    suf = jnp.concatenate([enc[1:], jnp.full((1, 1), BIG, jnp.int32)],
                          axis=0)
    s = 1
    while s < NB:
        suf = jnp.minimum(suf, jnp.concatenate(
            [suf[s:], jnp.full((s, 1), BIG, jnp.int32)], axis=0))
        s *= 2
    chg_ref[...] = chg
    slt_ref[...] = (r - 1) % 2
    nsl_ref[...] = r % 2
    hnx_ref[...] = (suf < BIG).astype(jnp.int32)
    nxe_ref[...] = jnp.where(suf < BIG, suf % 16, 0)


def _router(x, wr):
    return pl.pallas_call(
        _router_body,
        out_shape=(
            jax.ShapeDtypeStruct((T, H), jnp.int32),
            jax.ShapeDtypeStruct((T, 1), jnp.int32),
            jax.ShapeDtypeStruct((T, 1), jnp.int32),
            jax.ShapeDtypeStruct((T, 128), jnp.float32),
            jax.ShapeDtypeStruct((T, 128), jnp.float32),
            jax.ShapeDtypeStruct((NB, 1), jnp.int32),
            jax.ShapeDtypeStruct((NB, 1), jnp.int32),
            jax.ShapeDtypeStruct((NB, 1), jnp.int32),
            jax.ShapeDtypeStruct((NB, 1), jnp.int32),
            jax.ShapeDtypeStruct((NB, 1), jnp.int32),
            jax.ShapeDtypeStruct((NB, 1), jnp.int32),
            jax.ShapeDtypeStruct((NB, 1), jnp.int32),
            jax.ShapeDtypeStruct((1, 1), jnp.float32),
        ),
        out_specs=(
            pl.BlockSpec(memory_space=pltpu.VMEM),
            pl.BlockSpec(memory_space=pltpu.VMEM),
            pl.BlockSpec(memory_space=pltpu.VMEM),
            pl.BlockSpec(memory_space=pltpu.VMEM),
            pl.BlockSpec(memory_space=pltpu.VMEM),
            pl.BlockSpec(memory_space=pltpu.VMEM),
            pl.BlockSpec(memory_space=pltpu.VMEM),
            pl.BlockSpec(memory_space=pltpu.VMEM),
            pl.BlockSpec(memory_space=pltpu.VMEM),
            pl.BlockSpec(memory_space=pltpu.VMEM),
            pl.BlockSpec(memory_space=pltpu.VMEM),
            pl.BlockSpec(memory_space=pltpu.VMEM),
            pl.BlockSpec(memory_space=pltpu.SMEM),
        ),
    )(x, wr)


def _sc_mesh():
    return plsc.VectorSubcoreMesh(core_axis_name="c", subcore_axis_name="s",
                                  num_cores=2, num_subcores=16)


def _dispatch(xpk, slot0, slot1, w0w, w1w):
    @pl.kernel(
        out_type=(
            jax.ShapeDtypeStruct((S, H), jnp.int32),
            jax.ShapeDtypeStruct((S, 128), jnp.float32),
        ),
        mesh=_sc_mesh(),
        scratch_types=[
            pltpu.VMEM((1, TW), jnp.int32),
            pltpu.VMEM((1, TW), jnp.int32),
            pltpu.VMEM((TW, H), jnp.int32),
            pltpu.VMEM((TW, 128), jnp.float32),
            pltpu.VMEM((TW, 128), jnp.float32),
            pltpu.SemaphoreType.DMA,
            pltpu.SemaphoreType.DMA,
        ],
    )
    def disp(x_hbm, s0_hbm, s1_hbm, w0_hbm, w1_hbm, xs_hbm, ws_hbm,
             idx0, idx1, xv, wv0, wv1, lsem, ssem):
        wid = lax.axis_index("s") * 2 + lax.axis_index("c")
        base = wid * TW
        l0 = pltpu.async_copy(s0_hbm.at[pl.ds(base, TW)], idx0.at[0], lsem)
        l1 = pltpu.async_copy(s1_hbm.at[pl.ds(base, TW)], idx1.at[0], lsem)
        l2 = pltpu.async_copy(x_hbm.at[pl.ds(base, TW)], xv, lsem)
        l3 = pltpu.async_copy(w0_hbm.at[pl.ds(base, TW)], wv0, lsem)
        l4 = pltpu.async_copy(w1_hbm.at[pl.ds(base, TW)], wv1, lsem)
        l0.wait()
        l1.wait()
        l2.wait()
        l3.wait()
        l4.wait()
        s0 = pltpu.async_copy(xv, xs_hbm.at[idx0.at[0]], ssem)
        s1 = pltpu.async_copy(xv, xs_hbm.at[idx1.at[0]], ssem)
        s2 = pltpu.async_copy(wv0, ws_hbm.at[idx0.at[0]], ssem)
        s3 = pltpu.async_copy(wv1, ws_hbm.at[idx1.at[0]], ssem)
        s0.wait()
        s1.wait()
        s2.wait()
        s3.wait()

    return disp(xpk, slot0, slot1, w0w, w1w)


def _ffn_body(be_r, act_r, chg_r, slt_r, nsl_r, hnx_r, nxe_r,
              xs_ref, w1_hbm, w2_hbm, ws_ref, ys_ref, w1s, w2s, sw1, sw2):
    b = pl.program_id(0)

    def cp1(e, s):
        return pltpu.make_async_copy(w1_hbm.at[e], w1s.at[s], sw1.at[s])

    def cp2(e, s):
        return pltpu.make_async_copy(w2_hbm.at[e], w2s.at[s], sw2.at[s])

    @pl.when(b == 0)
    def _():
        cp1(be_r[0], 0).start()
        cp2(be_r[0], 0).start()

    @pl.when(chg_r[b] == 1)
    def _():
        s = slt_r[b]
        cp1(be_r[b], s).wait()
        cp2(be_r[b], s).wait()

        @pl.when(hnx_r[b] == 1)
        def _():
            cp1(nxe_r[b], nsl_r[b]).start()
            cp2(nxe_r[b], nsl_r[b]).start()

    @pl.when(act_r[b] > 0)
    def _():
        s = slt_r[b]
        xb = _unpack_halves(xs_ref[...])
        w1 = w1s[s].astype(jnp.bfloat16)
        h = jnp.dot(xb, w1, preferred_element_type=jnp.float32)
        h = jnp.maximum(h, 0.0).astype(jnp.bfloat16)
        w2 = w2s[s].astype(jnp.bfloat16)
        y = jnp.dot(h, w2, preferred_element_type=jnp.float32)
        ys_ref[...] = _pack_halves(y * ws_ref[:, 0:1])


def _ffn(be, act, chg, slt, nsl, hnx, nxe, xs, w1, w2, ws):
    return pl.pallas_call(
        _ffn_body,
        grid_spec=pltpu.PrefetchScalarGridSpec(
            num_scalar_prefetch=7,
            grid=(NB,),
            in_specs=[
                pl.BlockSpec((BLK, H), lambda b, *_: (b, 0)),
                pl.BlockSpec(memory_space=pl.ANY),
                pl.BlockSpec(memory_space=pl.ANY),
                pl.BlockSpec((BLK, 128), lambda b, *_: (b, 0)),
            ],
            out_specs=pl.BlockSpec((BLK, H), lambda b, *_: (b, 0)),
            scratch_shapes=[
                pltpu.VMEM((2, D, FF), jnp.float32),
                pltpu.VMEM((2, FF, D), jnp.float32),
                pltpu.SemaphoreType.DMA((2,)),
                pltpu.SemaphoreType.DMA((2,)),
            ],
        ),
        out_shape=jax.ShapeDtypeStruct((S, H), jnp.int32),
        compiler_params=pltpu.CompilerParams(
            dimension_semantics=("arbitrary",),
        ),
    )(be, act, chg, slt, nsl, hnx, nxe, xs, w1, w2, ws)


_CW = 16


def _combine(slot0, slot1, ys):
    nch = TW // _CW

    @pl.kernel(
        out_type=jax.ShapeDtypeStruct((T, D), jnp.float32),
        mesh=_sc_mesh(),
        scratch_types=[
            pltpu.VMEM((1, _CW), jnp.int32),
            pltpu.VMEM((1, _CW), jnp.int32),
            pltpu.VMEM((1, _CW), jnp.int32),
            pltpu.VMEM((1, _CW), jnp.int32),
            pltpu.VMEM((_CW, H), jnp.int32),
            pltpu.VMEM((_CW, H), jnp.int32),
            pltpu.VMEM((_CW, H), jnp.int32),
            pltpu.VMEM((_CW, H), jnp.int32),
            pltpu.VMEM((_CW, D), jnp.float32),
            pltpu.SemaphoreType.DMA,
            pltpu.SemaphoreType.DMA,
        ],
    )
    def comb(s0_hbm, s1_hbm, ys_hbm, out_hbm, idx0a, idx1a, idx0b, idx1b,
             g0a, g1a, g0b, g1b, ov, sema, semb):
        wid = lax.axis_index("s") * 2 + lax.axis_index("c")
        himask = jnp.int32(-65536)
        slots = ((idx0a, idx1a, g0a, g1a, sema),
                 (idx0b, idx1b, g0b, g1b, semb))

        def start(sc, slot):
            i0, i1, g0, g1, sem = slot
            base = wid * TW + sc * _CW
            pltpu.sync_copy(s0_hbm.at[pl.ds(base, _CW)], i0.at[0])
            pltpu.sync_copy(s1_hbm.at[pl.ds(base, _CW)], i1.at[0])
            cp0 = pltpu.async_copy(ys_hbm.at[i0.at[0]], g0, sem)
            cp1 = pltpu.async_copy(ys_hbm.at[i1.at[0]], g1, sem)
            return cp0, cp1

        pend = start(0, slots[0])
        for sc in range(nch):
            _, _, g0, g1, _ = slots[sc % 2]
            pend[0].wait()
            pend[1].wait()
            if sc + 1 < nch:
                pend = start(sc + 1, slots[(sc + 1) % 2])

            @pl.loop(0, _CW)
            def _(i):
                @plsc.parallel_loop(0, H, 16, unroll=4)
                def _(cc):
                    slc = (pl.ds(i, 1), pl.ds(cc, 16))
                    c0 = g0.at[*slc][...]
                    c1 = g1.at[*slc][...]
                    lo = (lax.bitcast_convert_type(c0 << 16, jnp.float32)
                          + lax.bitcast_convert_type(c1 << 16, jnp.float32))
                    hi = (lax.bitcast_convert_type(c0 & himask, jnp.float32)
                          + lax.bitcast_convert_type(c1 & himask, jnp.float32))
                    ov.at[pl.ds(i, 1), pl.ds(cc, 16)][...] = lo
                    ov.at[pl.ds(i, 1), pl.ds(H + cc, 16)][...] = hi

            pltpu.sync_copy(ov, out_hbm.at[pl.ds(wid * TW + sc * _CW, _CW)])

    return comb(slot0, slot1, ys)


def kernel(input_batch, W_router, W1, W2):
    x = input_batch
    (xpk, s0, s1, w0w, w1w, be, act, chg, slt, nsl, hnx, nxe,
     loss) = _router(x, W_router)
    slot0 = s0.reshape(T)
    slot1 = s1.reshape(T)
    xs, ws = _dispatch(xpk, slot0, slot1, w0w, w1w)
    ys = _ffn(be.reshape(NB), act.reshape(NB), chg.reshape(NB),
              slt.reshape(NB), nsl.reshape(NB), hnx.reshape(NB),
              nxe.reshape(NB), xs, W1, W2, ws)
    out = _combine(slot0, slot1, ys)
    return (out, loss.reshape(()))


# speedup vs baseline: 1.2699x; 1.0007x over previous
import jax
import jax.numpy as jnp
from jax import lax
from jax.experimental import pallas as pl
from jax.experimental.pallas import tpu as pltpu
from jax.experimental.pallas import tpu_sc as plsc

T, D, FF, E = 2048, 1024, 2048, 8
H = D // 2
BLK = 256
S = T * 2 + E * BLK
NB = S // BLK
NW = 32
TW = T // NW


def _pack_halves(a):
    b = a.astype(jnp.bfloat16)
    lo = lax.bitcast_convert_type(b[:, :H], jnp.uint16).astype(jnp.uint32)
    hi = lax.bitcast_convert_type(b[:, H:], jnp.uint16).astype(jnp.uint32)
    return (lo | (hi << 16)).astype(jnp.int32)


def _unpack_halves(p):
    lo = lax.bitcast_convert_type((p & 0xFFFF).astype(jnp.uint16),
                                  jnp.bfloat16)
    hi = lax.bitcast_convert_type(
        (p >> 16).astype(jnp.uint16), jnp.bfloat16)
    return jnp.concatenate([lo, hi], axis=1)


def _router_body(x_ref, wr_ref, xpk_ref, s0_ref, s1_ref, w0_ref, w1_ref,
                 be_ref, act_ref, chg_ref, slt_ref, nsl_ref, hnx_ref,
                 nxe_ref, loss_ref):
    x = x_ref[...]
    xpk_ref[...] = _pack_halves(x)
    logits = jnp.dot(x, wr_ref[...], preferred_element_type=jnp.float32)
    m = jnp.max(logits, axis=1, keepdims=True)
    ex = jnp.exp(logits - m)
    probs = ex / jnp.sum(ex, axis=1, keepdims=True)
    lane = lax.broadcasted_iota(jnp.int32, probs.shape, 1)
    m1 = jnp.max(probs, axis=1, keepdims=True)
    i1 = jnp.min(jnp.where(probs == m1, lane, E), axis=1, keepdims=True)
    sel1 = lane == i1
    probsm = jnp.where(sel1, -jnp.inf, probs)
    m2 = jnp.max(probsm, axis=1, keepdims=True)
    i2 = jnp.min(jnp.where(probsm == m2, lane, E), axis=1, keepdims=True)
    sel2 = lane == i2

    dmask = ((sel1 & (m1 > 0.0)) | (sel2 & (m2 > 0.0))).astype(jnp.float32)
    frac = jnp.mean(dmask, axis=0)
    meanp = jnp.mean(probs, axis=0)
    loss_ref[0, 0] = jnp.float32(E) * jnp.sum(frac * meanp)

    c1 = sel1.astype(jnp.int32)
    c2 = sel2.astype(jnp.int32)
    c = c1 + c2
    inc = c
    s = 1
    while s < T:
        inc = inc + jnp.concatenate(
            [jnp.zeros((s, E), jnp.int32), inc[:T - s]], axis=0)
        s *= 2
    excl = inc - c
    counts = jnp.sum(c, axis=0, keepdims=True)
    pc = ((counts + (BLK - 1)) // BLK) * BLK
    erow = lax.broadcasted_iota(jnp.int32, (E, E), 0)
    ecol = lax.broadcasted_iota(jnp.int32, (E, E), 1)
    tri = (erow < ecol).astype(jnp.float32)
    off = jnp.dot(pc.astype(jnp.float32), tri,
                  preferred_element_type=jnp.float32).astype(jnp.int32)

    rank0 = jnp.sum(c1 * excl, axis=1, keepdims=True)
    rank1 = jnp.sum(c2 * (excl + c1), axis=1, keepdims=True)
    off0 = jnp.sum(c1 * off, axis=1, keepdims=True)
    off1 = jnp.sum(c2 * off, axis=1, keepdims=True)
    s0_ref[...] = off0 + rank0
    s1_ref[...] = off1 + rank1
    w0_ref[...] = jnp.broadcast_to(m1, (T, 128))
    w1_ref[...] = jnp.broadcast_to(m2, (T, 128))

    ends = off + pc
    bstart = lax.broadcasted_iota(jnp.int32, (NB, E), 0) * BLK
    be = jnp.minimum(jnp.sum((bstart >= ends).astype(jnp.int32),
                             axis=1, keepdims=True), E - 1)
    oh = (lax.broadcasted_iota(jnp.int32, (NB, E), 1) == be).astype(jnp.int32)
    real_end = jnp.sum(oh * (off + counts), axis=1, keepdims=True)
    bcol = lax.broadcasted_iota(jnp.int32, (NB, 1), 0) * BLK
    be_ref[...] = be
    act_ref[...] = (real_end > bcol).astype(jnp.int32)

    bprev = jnp.concatenate([be[:1], be[:NB - 1]], axis=0)
    biota = lax.broadcasted_iota(jnp.int32, (NB, 1), 0)
    chg = ((biota == 0) | (be != bprev)).astype(jnp.int32)
    r = chg
    s = 1
    while s < NB:
        r = r + jnp.concatenate(
            [jnp.zeros((s, 1), jnp.int32), r[:NB - s]], axis=0)
        s *= 2
    BIG = jnp.int32(1 << 30)
    enc = jnp.where(chg == 1, biota * 16 + be, BIG)
    suf = jnp.concatenate([enc[1:], jnp.full((1, 1), BIG, jnp.int32)],
                          axis=0)
    s = 1
    while s < NB:
        suf = jnp.minimum(suf, jnp.concatenate(
            [suf[s:], jnp.full((s, 1), BIG, jnp.int32)], axis=0))
        s *= 2
    chg_ref[...] = chg
    slt_ref[...] = (r - 1) % 2
    nsl_ref[...] = r % 2
    hnx_ref[...] = (suf < BIG).astype(jnp.int32)
    nxe_ref[...] = jnp.where(suf < BIG, suf % 16, 0)


def _router(x, wr):
    return pl.pallas_call(
        _router_body,
        out_shape=(
            jax.ShapeDtypeStruct((T, H), jnp.int32),
            jax.ShapeDtypeStruct((T, 1), jnp.int32),
            jax.ShapeDtypeStruct((T, 1), jnp.int32),
            jax.ShapeDtypeStruct((T, 128), jnp.float32),
            jax.ShapeDtypeStruct((T, 128), jnp.float32),
            jax.ShapeDtypeStruct((NB, 1), jnp.int32),
            jax.ShapeDtypeStruct((NB, 1), jnp.int32),
            jax.ShapeDtypeStruct((NB, 1), jnp.int32),
            jax.ShapeDtypeStruct((NB, 1), jnp.int32),
            jax.ShapeDtypeStruct((NB, 1), jnp.int32),
            jax.ShapeDtypeStruct((NB, 1), jnp.int32),
            jax.ShapeDtypeStruct((NB, 1), jnp.int32),
            jax.ShapeDtypeStruct((1, 1), jnp.float32),
        ),
        out_specs=(
            pl.BlockSpec(memory_space=pltpu.VMEM),
            pl.BlockSpec(memory_space=pltpu.VMEM),
            pl.BlockSpec(memory_space=pltpu.VMEM),
            pl.BlockSpec(memory_space=pltpu.VMEM),
            pl.BlockSpec(memory_space=pltpu.VMEM),
            pl.BlockSpec(memory_space=pltpu.VMEM),
            pl.BlockSpec(memory_space=pltpu.VMEM),
            pl.BlockSpec(memory_space=pltpu.VMEM),
            pl.BlockSpec(memory_space=pltpu.VMEM),
            pl.BlockSpec(memory_space=pltpu.VMEM),
            pl.BlockSpec(memory_space=pltpu.VMEM),
            pl.BlockSpec(memory_space=pltpu.VMEM),
            pl.BlockSpec(memory_space=pltpu.SMEM),
        ),
    )(x, wr)


def _sc_mesh():
    return plsc.VectorSubcoreMesh(core_axis_name="c", subcore_axis_name="s",
                                  num_cores=2, num_subcores=16)


def _dispatch(xpk, slot0, slot1, w0w, w1w):
    @pl.kernel(
        out_type=(
            jax.ShapeDtypeStruct((S, H), jnp.int32),
            jax.ShapeDtypeStruct((S, 128), jnp.float32),
        ),
        mesh=_sc_mesh(),
        scratch_types=[
            pltpu.VMEM((1, TW), jnp.int32),
            pltpu.VMEM((1, TW), jnp.int32),
            pltpu.VMEM((TW, H), jnp.int32),
            pltpu.VMEM((TW, 128), jnp.float32),
            pltpu.VMEM((TW, 128), jnp.float32),
            pltpu.SemaphoreType.DMA,
            pltpu.SemaphoreType.DMA,
        ],
    )
    def disp(x_hbm, s0_hbm, s1_hbm, w0_hbm, w1_hbm, xs_hbm, ws_hbm,
             idx0, idx1, xv, wv0, wv1, lsem, ssem):
        wid = lax.axis_index("s") * 2 + lax.axis_index("c")
        base = wid * TW
        l0 = pltpu.async_copy(s0_hbm.at[pl.ds(base, TW)], idx0.at[0], lsem)
        l1 = pltpu.async_copy(s1_hbm.at[pl.ds(base, TW)], idx1.at[0], lsem)
        l2 = pltpu.async_copy(x_hbm.at[pl.ds(base, TW)], xv, lsem)
        l3 = pltpu.async_copy(w0_hbm.at[pl.ds(base, TW)], wv0, lsem)
        l4 = pltpu.async_copy(w1_hbm.at[pl.ds(base, TW)], wv1, lsem)
        l0.wait()
        l1.wait()
        l2.wait()
        l3.wait()
        l4.wait()
        s0 = pltpu.async_copy(xv, xs_hbm.at[idx0.at[0]], ssem)
        s1 = pltpu.async_copy(xv, xs_hbm.at[idx1.at[0]], ssem)
        s2 = pltpu.async_copy(wv0, ws_hbm.at[idx0.at[0]], ssem)
        s3 = pltpu.async_copy(wv1, ws_hbm.at[idx1.at[0]], ssem)
        s0.wait()
        s1.wait()
        s2.wait()
        s3.wait()

    return disp(xpk, slot0, slot1, w0w, w1w)


def _ffn_body(be_r, act_r, chg_r, slt_r, nsl_r, hnx_r, nxe_r,
              xs_ref, w1_hbm, w2_hbm, ws_ref, ys_ref, w1s, w2s, sw1, sw2):
    b = pl.program_id(0)

    def cp1(e, s):
        return pltpu.make_async_copy(w1_hbm.at[e], w1s.at[s], sw1.at[s])

    def cp2(e, s):
        return pltpu.make_async_copy(w2_hbm.at[e], w2s.at[s], sw2.at[s])

    @pl.when(b == 0)
    def _():
        cp1(be_r[0], 0).start()
        cp2(be_r[0], 0).start()

    @pl.when(chg_r[b] == 1)
    def _():
        s = slt_r[b]
        cp1(be_r[b], s).wait()
        cp2(be_r[b], s).wait()

        @pl.when(hnx_r[b] == 1)
        def _():
            cp1(nxe_r[b], nsl_r[b]).start()
            cp2(nxe_r[b], nsl_r[b]).start()

    @pl.when(act_r[b] > 0)
    def _():
        s = slt_r[b]
        xb = _unpack_halves(xs_ref[...])
        w1 = w1s[s].astype(jnp.bfloat16)
        h = jnp.dot(xb, w1, preferred_element_type=jnp.float32)
        h = jnp.maximum(h, 0.0).astype(jnp.bfloat16)
        w2 = w2s[s].astype(jnp.bfloat16)
        y = jnp.dot(h, w2, preferred_element_type=jnp.float32)
        ys_ref[...] = _pack_halves(y * ws_ref[:, 0:1])


def _ffn(be, act, chg, slt, nsl, hnx, nxe, xs, w1, w2, ws):
    return pl.pallas_call(
        _ffn_body,
        grid_spec=pltpu.PrefetchScalarGridSpec(
            num_scalar_prefetch=7,
            grid=(NB,),
            in_specs=[
                pl.BlockSpec((BLK, H), lambda b, *_: (b, 0)),
                pl.BlockSpec(memory_space=pl.ANY),
                pl.BlockSpec(memory_space=pl.ANY),
                pl.BlockSpec((BLK, 128), lambda b, *_: (b, 0)),
            ],
            out_specs=pl.BlockSpec((BLK, H), lambda b, *_: (b, 0)),
            scratch_shapes=[
                pltpu.VMEM((2, D, FF), jnp.float32),
                pltpu.VMEM((2, FF, D), jnp.float32),
                pltpu.SemaphoreType.DMA((2,)),
                pltpu.SemaphoreType.DMA((2,)),
            ],
        ),
        out_shape=jax.ShapeDtypeStruct((S, H), jnp.int32),
        compiler_params=pltpu.CompilerParams(
            dimension_semantics=("arbitrary",),
        ),
    )(be, act, chg, slt, nsl, hnx, nxe, xs, w1, w2, ws)


_CW = 16


def _combine(slot0, slot1, ys):
    nch = TW // _CW

    @pl.kernel(
        out_type=jax.ShapeDtypeStruct((T, D), jnp.float32),
        mesh=_sc_mesh(),
        scratch_types=[
            pltpu.VMEM((1, _CW), jnp.int32),
            pltpu.VMEM((1, _CW), jnp.int32),
            pltpu.VMEM((1, _CW), jnp.int32),
            pltpu.VMEM((1, _CW), jnp.int32),
            pltpu.VMEM((_CW, H), jnp.int32),
            pltpu.VMEM((_CW, H), jnp.int32),
            pltpu.VMEM((_CW, H), jnp.int32),
            pltpu.VMEM((_CW, H), jnp.int32),
            pltpu.VMEM((_CW, D), jnp.float32),
            pltpu.SemaphoreType.DMA,
            pltpu.SemaphoreType.DMA,
        ],
    )
    def comb(s0_hbm, s1_hbm, ys_hbm, out_hbm, idx0a, idx1a, idx0b, idx1b,
             g0a, g1a, g0b, g1b, ov, sema, semb):
        wid = lax.axis_index("s") * 2 + lax.axis_index("c")
        himask = jnp.int32(-65536)
        slots = ((idx0a, idx1a, g0a, g1a, sema),
                 (idx0b, idx1b, g0b, g1b, semb))

        def start(sc, slot):
            i0, i1, g0, g1, sem = slot
            base = wid * TW + sc * _CW
            pltpu.sync_copy(s0_hbm.at[pl.ds(base, _CW)], i0.at[0])
            pltpu.sync_copy(s1_hbm.at[pl.ds(base, _CW)], i1.at[0])
            cp0 = pltpu.async_copy(ys_hbm.at[i0.at[0]], g0, sem)
            cp1 = pltpu.async_copy(ys_hbm.at[i1.at[0]], g1, sem)
            return cp0, cp1

        pend = start(0, slots[0])
        for sc in range(nch):
            _, _, g0, g1, _ = slots[sc % 2]
            pend[0].wait()
            pend[1].wait()
            if sc + 1 < nch:
                pend = start(sc + 1, slots[(sc + 1) % 2])

            @pl.loop(0, _CW)
            def _(i):
                @plsc.parallel_loop(0, H, 16, unroll=8)
                def _(cc):
                    slc = (pl.ds(i, 1), pl.ds(cc, 16))
                    c0 = g0.at[*slc][...]
                    c1 = g1.at[*slc][...]
                    lo = (lax.bitcast_convert_type(c0 << 16, jnp.float32)
                          + lax.bitcast_convert_type(c1 << 16, jnp.float32))
                    hi = (lax.bitcast_convert_type(c0 & himask, jnp.float32)
                          + lax.bitcast_convert_type(c1 & himask, jnp.float32))
                    ov.at[pl.ds(i, 1), pl.ds(cc, 16)][...] = lo
                    ov.at[pl.ds(i, 1), pl.ds(H + cc, 16)][...] = hi

            pltpu.sync_copy(ov, out_hbm.at[pl.ds(wid * TW + sc * _CW, _CW)])

    return comb(slot0, slot1, ys)


def kernel(input_batch, W_router, W1, W2):
    x = input_batch
    (xpk, s0, s1, w0w, w1w, be, act, chg, slt, nsl, hnx, nxe,
     loss) = _router(x, W_router)
    slot0 = s0.reshape(T)
    slot1 = s1.reshape(T)
    xs, ws = _dispatch(xpk, slot0, slot1, w0w, w1w)
    ys = _ffn(be.reshape(NB), act.reshape(NB), chg.reshape(NB),
              slt.reshape(NB), nsl.reshape(NB), hnx.reshape(NB),
              nxe.reshape(NB), xs, W1, W2, ws)
    out = _combine(slot0, slot1, ys)
    return (out, loss.reshape(()))
